# speedup vs baseline: 8.1371x; 8.1371x over previous
import functools

import jax
import jax.numpy as jnp
from jax import lax
from jax.experimental import pallas as pl
from jax.experimental.pallas import tpu as pltpu
from jax.experimental.pallas import tpu_sc as plsc


@functools.lru_cache(maxsize=None)
def _build(N, V, D, NC, NS):
    NW = NC * NS
    per_w = N // NW
    C = 512
    iters = per_w // C

    mesh = plsc.VectorSubcoreMesh(core_axis_name="c", subcore_axis_name="s")

    @functools.partial(
        pl.kernel,
        mesh=mesh,
        out_type=jax.ShapeDtypeStruct((N, D), jnp.float32),
        scratch_types=[
            pltpu.VMEM((C,), jnp.int32),
            pltpu.VMEM((C, D), jnp.float32),
            pltpu.SemaphoreType.DMA,
        ],
    )
    def gather_kernel(ids_hbm, table_hbm, out_hbm, idx_v, rows_v, sem):
        wid = lax.axis_index("s") * NC + lax.axis_index("c")
        base = wid * per_w

        def body(i, carry):
            off = base + i * C
            pltpu.sync_copy(ids_hbm.at[pl.ds(off, C)], idx_v)
            pltpu.async_copy(table_hbm.at[idx_v], rows_v, sem).wait()
            pltpu.sync_copy(rows_v, out_hbm.at[pl.ds(off, C)])
            return carry

        lax.fori_loop(0, iters, body, 0)

    return gather_kernel


def kernel(input_ids, embedding_matrix):
    B, S = input_ids.shape
    V, D = embedding_matrix.shape
    N = B * S
    info = plsc.get_sparse_core_info()
    fn = _build(N, V, D, info.num_cores, info.num_subcores)
    out = fn(input_ids.reshape(N), embedding_matrix)
    return out.reshape(B, S, D)


# speedup vs baseline: 8.9419x; 1.0989x over previous
import functools

import jax
import jax.numpy as jnp
from jax import lax
from jax.experimental import pallas as pl
from jax.experimental.pallas import tpu as pltpu
from jax.experimental.pallas import tpu_sc as plsc


@functools.lru_cache(maxsize=None)
def _build(N, V, D, NC, NS):
    NW = NC * NS
    per_w = N // NW
    C = 400
    iters = per_w // C
    half = iters // 2

    mesh = plsc.VectorSubcoreMesh(core_axis_name="c", subcore_axis_name="s")

    @functools.partial(
        pl.kernel,
        mesh=mesh,
        out_type=jax.ShapeDtypeStruct((N, D), jnp.float32),
        scratch_types=[
            pltpu.VMEM((C,), jnp.int32),
            pltpu.VMEM((C,), jnp.int32),
            pltpu.VMEM((C, D), jnp.float32),
            pltpu.VMEM((C, D), jnp.float32),
            pltpu.SemaphoreType.DMA,
            pltpu.SemaphoreType.DMA,
            pltpu.SemaphoreType.DMA,
            pltpu.SemaphoreType.DMA,
        ],
    )
    def gather_kernel(ids_hbm, table_hbm, out_hbm, idx0, idx1, rows0, rows1,
                      sg0, sg1, ss0, ss1):
        wid = lax.axis_index("s") * NC + lax.axis_index("c")
        base = wid * per_w

        pltpu.sync_copy(ids_hbm.at[pl.ds(base, C)], idx0)
        pltpu.async_copy(table_hbm.at[idx0], rows0, sg0)

        def body(g, carry):
            off0 = base + (2 * g) * C
            off1 = off0 + C

            pltpu.make_async_copy(table_hbm.at[idx0], rows0, sg0).wait()
            pltpu.async_copy(rows0, out_hbm.at[pl.ds(off0, C)], ss0)

            @pl.when(g > 0)
            def _():
                pltpu.make_async_copy(
                    rows1, out_hbm.at[pl.ds(off1, C)], ss1).wait()

            pltpu.sync_copy(ids_hbm.at[pl.ds(off1, C)], idx1)
            pltpu.async_copy(table_hbm.at[idx1], rows1, sg1)
            pltpu.make_async_copy(table_hbm.at[idx1], rows1, sg1).wait()
            pltpu.async_copy(rows1, out_hbm.at[pl.ds(off1, C)], ss1)

            pltpu.make_async_copy(rows0, out_hbm.at[pl.ds(off0, C)], ss0).wait()

            @pl.when(g + 1 < half)
            def _():
                off2 = off0 + 2 * C
                pltpu.sync_copy(ids_hbm.at[pl.ds(off2, C)], idx0)
                pltpu.async_copy(table_hbm.at[idx0], rows0, sg0)

            return carry

        lax.fori_loop(0, half, body, 0)

        pltpu.make_async_copy(
            rows1, out_hbm.at[pl.ds(base + (iters - 1) * C, C)], ss1).wait()

    return gather_kernel


def kernel(input_ids, embedding_matrix):
    B, S = input_ids.shape
    V, D = embedding_matrix.shape
    N = B * S
    info = plsc.get_sparse_core_info()
    fn = _build(N, V, D, info.num_cores, info.num_subcores)
    out = fn(input_ids.reshape(N), embedding_matrix)
    return out.reshape(B, S, D)


# speedup vs baseline: 9.2488x; 1.0343x over previous
import functools

import jax
import jax.numpy as jnp
from jax import lax
from jax.experimental import pallas as pl
from jax.experimental.pallas import tpu as pltpu
from jax.experimental.pallas import tpu_sc as plsc

_NBUF = 4


@functools.lru_cache(maxsize=None)
def _build(N, V, D, NC, NS):
    NW = NC * NS
    per_w = N // NW
    C = 200
    iters = per_w // C
    assert iters % _NBUF == 0 and iters >= 2 * _NBUF

    mesh = plsc.VectorSubcoreMesh(core_axis_name="c", subcore_axis_name="s")

    scratch = (
        [pltpu.VMEM((C,), jnp.int32) for _ in range(_NBUF)]
        + [pltpu.VMEM((C, D), jnp.float32) for _ in range(_NBUF)]
        + [pltpu.SemaphoreType.DMA for _ in range(2 * _NBUF)]
    )

    @functools.partial(
        pl.kernel,
        mesh=mesh,
        out_type=jax.ShapeDtypeStruct((N, D), jnp.float32),
        scratch_types=scratch,
    )
    def gather_kernel(ids_hbm, table_hbm, out_hbm, *bufs):
        idx = bufs[0:_NBUF]
        rows = bufs[_NBUF:2 * _NBUF]
        sg = bufs[2 * _NBUF:3 * _NBUF]
        ss = bufs[3 * _NBUF:4 * _NBUF]

        wid = lax.axis_index("s") * NC + lax.axis_index("c")
        base = wid * per_w

        def start_gather(b, off):
            pltpu.sync_copy(ids_hbm.at[pl.ds(off, C)], idx[b])
            pltpu.async_copy(table_hbm.at[idx[b]], rows[b], sg[b])

        def wait_gather(b):
            pltpu.make_async_copy(table_hbm.at[idx[b]], rows[b], sg[b]).wait()

        def start_store(b, off):
            pltpu.async_copy(rows[b], out_hbm.at[pl.ds(off, C)], ss[b])

        def wait_store(b, off):
            pltpu.make_async_copy(rows[b], out_hbm.at[pl.ds(off, C)], ss[b]).wait()

        for b in range(_NBUF):
            start_gather(b, base + b * C)
        for b in range(2):
            wait_gather(b)
            start_store(b, base + b * C)

        def body(g, carry):
            first = base + (g + 1) * (_NBUF * C)
            for b in range(_NBUF):
                off = first + b * C
                wait_store(b, off - _NBUF * C)
                start_gather(b, off)
                jb = (b + 2) % _NBUF
                joff = off - 2 * C
                wait_gather(jb)
                start_store(jb, joff)
            return carry

        lax.fori_loop(0, iters // _NBUF - 1, body, 0)

        last = base + iters * C
        for i in (iters - 2, iters - 1):
            b = i % _NBUF
            wait_gather(b)
            start_store(b, base + i * C)
        for i in range(iters - _NBUF, iters):
            b = i % _NBUF
            wait_store(b, base + i * C)

    return gather_kernel


def kernel(input_ids, embedding_matrix):
    B, S = input_ids.shape
    V, D = embedding_matrix.shape
    N = B * S
    info = plsc.get_sparse_core_info()
    fn = _build(N, V, D, info.num_cores, info.num_subcores)
    out = fn(input_ids.reshape(N), embedding_matrix)
    return out.reshape(B, S, D)
